# baseline (speedup 1.0000x reference)
import functools

import jax
import jax.numpy as jnp
from jax import lax
from jax.experimental import pallas as pl
from jax.experimental.pallas import tpu as pltpu
from jax.experimental.pallas import tpu_sc as plsc

_NC = 2
_NS = 16
_NW = _NC * _NS

_CHUNK = 128
_CHUNKS_PER_BUF = 4
_ROWS_PER_BUF = _CHUNK * _CHUNKS_PER_BUF


def _make_gather(n_rows: int, embed: int):
    assert n_rows % (_NW * _ROWS_PER_BUF) == 0
    rows_per_w = n_rows // _NW
    n_chunks = rows_per_w // _CHUNK
    n_steps = rows_per_w // _ROWS_PER_BUF
    assert n_steps % 2 == 0 and n_steps >= 4
    mesh = plsc.VectorSubcoreMesh(core_axis_name="c", subcore_axis_name="s")

    @functools.partial(
        pl.kernel,
        out_type=jax.ShapeDtypeStruct((n_rows, embed), jnp.float32),
        mesh=mesh,
        compiler_params=pltpu.CompilerParams(use_tc_tiling_on_sc=False),
        scratch_types=[
            pltpu.VMEM((n_chunks, _CHUNK), jnp.int32),
            pltpu.VMEM((_ROWS_PER_BUF, embed), jnp.float32),
            pltpu.VMEM((_ROWS_PER_BUF, embed), jnp.float32),
            pltpu.SemaphoreType.DMA,
            pltpu.SemaphoreType.DMA,
        ],
    )
    def body(table_hbm, idx_hbm, out_hbm, idx_v, buf0, buf1, sem0, sem1):
        wid = lax.axis_index("s") * _NC + lax.axis_index("c")
        base = wid * rows_per_w

        pltpu.sync_copy(idx_hbm.at[wid], idx_v)

        def fire(step, buf, sem):
            for j in range(_CHUNKS_PER_BUF):
                pltpu.async_copy(
                    table_hbm.at[idx_v.at[step * _CHUNKS_PER_BUF + j]],
                    buf.at[pl.ds(j * _CHUNK, _CHUNK)],
                    sem,
                )

        def drain(step, buf, sem):
            for j in range(_CHUNKS_PER_BUF):
                pltpu.make_async_copy(
                    table_hbm.at[idx_v.at[step * _CHUNKS_PER_BUF + j]],
                    buf.at[pl.ds(j * _CHUNK, _CHUNK)],
                    sem,
                ).wait()

        def flush(step, buf):
            pltpu.sync_copy(
                buf, out_hbm.at[pl.ds(base + step * _ROWS_PER_BUF, _ROWS_PER_BUF)]
            )

        fire(0, buf0, sem0)

        def loop_body(t, carry):
            g = 2 * t
            fire(g + 1, buf1, sem1)
            drain(g, buf0, sem0)
            flush(g, buf0)
            fire(g + 2, buf0, sem0)
            drain(g + 1, buf1, sem1)
            flush(g + 1, buf1)
            return carry

        lax.fori_loop(0, n_steps // 2 - 1, loop_body, 0)

        g = n_steps - 2
        fire(g + 1, buf1, sem1)
        drain(g, buf0, sem0)
        flush(g, buf0)
        drain(g + 1, buf1, sem1)
        flush(g + 1, buf1)

    return body


def kernel(x, table):
    batch, fields = x.shape
    vocab, embed = table.shape
    n_rows = batch * fields
    idx = x.reshape(_NW, n_rows // _NW // _CHUNK, _CHUNK).astype(jnp.int32)
    out = _make_gather(n_rows, embed)(table, idx)
    return out.reshape(batch, fields, embed)
